# baseline (speedup 1.0000x reference)
import functools

import jax
import jax.numpy as jnp
from jax import lax
from jax.experimental import pallas as pl
from jax.experimental.pallas import tpu as pltpu
from jax.experimental.pallas import tpu_sc as plsc

N_ROWS = 100000
N_CLASSES = 128
CHUNK = 160
FLAT = CHUNK * N_CLASSES
N_CHUNKS = N_ROWS // CHUNK
GROUPS = CHUNK // 16
NBUF = 2

_info = plsc.get_sparse_core_info()
NC, NS = _info.num_cores, _info.num_subcores
NW = NC * NS
MAX_CHUNKS_PER_W = (N_CHUNKS + NW - 1) // NW
SUPER = MAX_CHUNKS_PER_W // NBUF


@functools.partial(
    pl.kernel,
    mesh=plsc.VectorSubcoreMesh(core_axis_name="c", subcore_axis_name="s"),
    out_type=jax.ShapeDtypeStruct((N_ROWS * N_CLASSES,), jnp.float32),
    scratch_types=[
        pltpu.VMEM((MAX_CHUNKS_PER_W * CHUNK,), jnp.int32),
        pltpu.VMEM((FLAT,), jnp.float32),
        pltpu.VMEM((FLAT,), jnp.float32),
        pltpu.SemaphoreType.DMA,
        pltpu.SemaphoreType.DMA,
        pltpu.SemaphoreType.DMA,
    ],
    compiler_params=pltpu.CompilerParams(needs_layout_passes=False),
)
def _one_hot_sc(idx_hbm, out_hbm, idx_v, b0, b1, sem_i, s0, s1):
    wid = lax.axis_index("s") * NC + lax.axis_index("c")
    sems = [s0, s1]
    bufs = [b0, b1]
    zeros = jnp.zeros((16,), jnp.float32)
    ones = jnp.ones((16,), jnp.float32)
    lane128 = lax.iota(jnp.int32, 16) * N_CLASSES

    def _idx_fetch(t, carry):
        cid = jnp.minimum(t * NW + wid, N_CHUNKS - 1)
        pltpu.async_copy(
            idx_hbm.at[pl.ds(cid * CHUNK, CHUNK)],
            idx_v.at[pl.ds(t * CHUNK, CHUNK)],
            sem_i,
        )
        return carry

    lax.fori_loop(0, MAX_CHUNKS_PER_W, _idx_fetch, 0)

    def _memset(buf):
        def _row(r, carry):
            for j in range(N_CLASSES // 16):
                buf[pl.ds(r * N_CLASSES + j * 16, 16)] = zeros
            return carry

        lax.fori_loop(0, CHUNK, _row, 0)

    def _idx_drain(t, carry):
        pltpu.make_async_copy(
            idx_hbm.at[pl.ds(0, CHUNK)], idx_v.at[pl.ds(0, CHUNK)], sem_i
        ).wait()
        return carry

    def _scatter(buf, t, val):
        def _g(g, carry):
            cols = idx_v[pl.ds(t * CHUNK + g * 16, 16)]
            plsc.store_scatter(buf, [lane128 + g * (16 * N_CLASSES) + cols], val)
            return carry

        lax.fori_loop(0, GROUPS, _g, 0, unroll=2)

    _memset(b0)
    lax.fori_loop(0, MAX_CHUNKS_PER_W, _idx_drain, 0)

    _scatter(b0, 0, ones)
    pltpu.async_copy(b0, out_hbm.at[pl.ds(wid * FLAT, FLAT)], s0)
    _memset(b1)
    _scatter(b1, 1, ones)
    pltpu.async_copy(b1, out_hbm.at[pl.ds((NW + wid) * FLAT, FLAT)], s1)

    def _super_body(s, carry):
        for b in range(NBUF):
            t = s * NBUF + b
            cid = t * NW + wid

            @pl.when(cid < N_CHUNKS)
            def _():
                pltpu.make_async_copy(
                    bufs[b], out_hbm.at[pl.ds(0, FLAT)], sems[b]
                ).wait()
                _scatter(bufs[b], t - NBUF, zeros)
                _scatter(bufs[b], t, ones)
                pltpu.async_copy(
                    bufs[b], out_hbm.at[pl.ds(cid * FLAT, FLAT)], sems[b]
                )

        return carry

    lax.fori_loop(1, SUPER, _super_body, 0)

    for b in range(NBUF):
        pltpu.make_async_copy(
            bufs[b], out_hbm.at[pl.ds(0, FLAT)], sems[b]
        ).wait()


def kernel(input):
    idx = jnp.reshape(input, (N_ROWS,))
    return jnp.reshape(_one_hot_sc(idx), (N_ROWS, N_CLASSES))


# speedup vs baseline: 1.0021x; 1.0021x over previous
import functools

import jax
import jax.numpy as jnp
from jax import lax
from jax.experimental import pallas as pl
from jax.experimental.pallas import tpu as pltpu
from jax.experimental.pallas import tpu_sc as plsc

N_ROWS = 100000
N_CLASSES = 128
CHUNK = 160
FLAT = CHUNK * N_CLASSES
N_CHUNKS = N_ROWS // CHUNK
GROUPS = CHUNK // 16
NBUF = 2

_info = plsc.get_sparse_core_info()
NC, NS = _info.num_cores, _info.num_subcores
NW = NC * NS
MAX_CHUNKS_PER_W = (N_CHUNKS + NW - 1) // NW
SUPER = MAX_CHUNKS_PER_W // NBUF


@functools.partial(
    pl.kernel,
    mesh=plsc.VectorSubcoreMesh(core_axis_name="c", subcore_axis_name="s"),
    out_type=jax.ShapeDtypeStruct((N_ROWS * N_CLASSES,), jnp.float32),
    scratch_types=[
        pltpu.VMEM((MAX_CHUNKS_PER_W * CHUNK,), jnp.int32),
        pltpu.VMEM((FLAT,), jnp.float32),
        pltpu.VMEM((FLAT,), jnp.float32),
        pltpu.SemaphoreType.DMA,
        pltpu.SemaphoreType.DMA,
        pltpu.SemaphoreType.DMA,
    ],
    compiler_params=pltpu.CompilerParams(needs_layout_passes=False),
)
def _one_hot_sc(idx_hbm, out_hbm, idx_v, b0, b1, sem_i, s0, s1):
    wid = lax.axis_index("s") * NC + lax.axis_index("c")
    sems = [s0, s1]
    bufs = [b0, b1]
    zeros = jnp.zeros((16,), jnp.float32)
    ones = jnp.ones((16,), jnp.float32)
    lane128 = lax.iota(jnp.int32, 16) * N_CLASSES

    def _idx_fetch(t, carry):
        cid = jnp.minimum(t * NW + wid, N_CHUNKS - 1)
        pltpu.async_copy(
            idx_hbm.at[pl.ds(cid * CHUNK, CHUNK)],
            idx_v.at[pl.ds(t * CHUNK, CHUNK)],
            sem_i,
        )
        return carry

    lax.fori_loop(0, MAX_CHUNKS_PER_W, _idx_fetch, 0)

    def _memset(buf):
        def _row(r, carry):
            for j in range(N_CLASSES // 16):
                buf[pl.ds(r * N_CLASSES + j * 16, 16)] = zeros
            return carry

        lax.fori_loop(0, CHUNK, _row, 0)

    def _idx_drain(t, carry):
        pltpu.make_async_copy(
            idx_hbm.at[pl.ds(0, CHUNK)], idx_v.at[pl.ds(0, CHUNK)], sem_i
        ).wait()
        return carry

    def _scatter(buf, t, val):
        def _g(g, carry):
            cols = idx_v[pl.ds(t * CHUNK + g * 16, 16)]
            plsc.store_scatter(buf, [lane128 + g * (16 * N_CLASSES) + cols], val)
            return carry

        lax.fori_loop(0, GROUPS, _g, 0)

    _memset(b0)
    lax.fori_loop(0, MAX_CHUNKS_PER_W, _idx_drain, 0)

    _scatter(b0, 0, ones)
    pltpu.async_copy(b0, out_hbm.at[pl.ds(wid * FLAT, FLAT)], s0)
    _memset(b1)
    _scatter(b1, 1, ones)
    pltpu.async_copy(b1, out_hbm.at[pl.ds((NW + wid) * FLAT, FLAT)], s1)

    def _super_body(s, carry):
        for b in range(NBUF):
            t = s * NBUF + b
            cid = t * NW + wid

            @pl.when(cid < N_CHUNKS)
            def _():
                pltpu.make_async_copy(
                    bufs[b], out_hbm.at[pl.ds(0, FLAT)], sems[b]
                ).wait()
                _scatter(bufs[b], t - NBUF, zeros)
                _scatter(bufs[b], t, ones)
                pltpu.async_copy(
                    bufs[b], out_hbm.at[pl.ds(cid * FLAT, FLAT)], sems[b]
                )

        return carry

    lax.fori_loop(1, SUPER, _super_body, 0)

    for b in range(NBUF):
        pltpu.make_async_copy(
            bufs[b], out_hbm.at[pl.ds(0, FLAT)], sems[b]
        ).wait()


def kernel(input):
    idx = jnp.reshape(input, (N_ROWS,))
    return jnp.reshape(_one_hot_sc(idx), (N_ROWS, N_CLASSES))
